# Initial kernel scaffold; baseline (speedup 1.0000x reference)
import jax
import jax.numpy as jnp
from jax.experimental import pallas as pl


def kernel(x, edge_index, W_in, b_in, W_h, b_h, W_out, b_out):
    raise NotImplementedError("write your pallas kernel here")


# speedup vs baseline: 6.3630x; 6.3630x over previous
import functools

import jax
import jax.numpy as jnp
from jax import lax
from jax.experimental import pallas as pl
from jax.experimental.pallas import tpu as pltpu
from jax.experimental.pallas import tpu_sc as plsc

_NC = 2
_NS = 16
_LANES = 16


@functools.lru_cache(maxsize=None)
def _make_deg(n, e):
    nw = _NC * _NS
    ew = e // nw
    b = 80
    assert ew % b == 0 and e % nw == 0
    nchunk = ew // b
    npad = -(-n // 128) * 128
    rt = npad // _NS
    zr = 128
    w = 128
    mesh = plsc.VectorSubcoreMesh(core_axis_name="c", subcore_axis_name="s",
                                  num_cores=_NC, num_subcores=_NS)

    @functools.partial(
        pl.kernel,
        out_type=jax.ShapeDtypeStruct((_NC, npad, w), jnp.float32),
        mesh=mesh,
        scratch_types=[
            pltpu.VMEM((b,), jnp.int32),
            pltpu.VMEM((b, w), jnp.float32),
            pltpu.VMEM((zr, w), jnp.float32),
            pltpu.VMEM_SHARED((npad, w), jnp.float32),
        ],
    )
    def deg_kernel(src_hbm, ones_hbm, zeros_hbm, out_hbm, sidx, ones_v, zbuf,
                   acc):
        c = lax.axis_index("c")
        s = lax.axis_index("s")
        wid = s * _NC + c

        pltpu.sync_copy(ones_hbm, ones_v)
        pltpu.sync_copy(zeros_hbm, zbuf)
        for r in range(rt // zr):
            pltpu.sync_copy(zbuf, acc.at[pl.ds(s * rt + r * zr, zr)])
        if rt % zr:
            pltpu.sync_copy(zbuf.at[pl.ds(0, rt % zr)],
                            acc.at[pl.ds(s * rt + (rt // zr) * zr, rt % zr)])
        plsc.subcore_barrier()

        base = wid * ew

        def chunk(ci, _):
            off = pl.multiple_of(base + ci * b, 8)
            pltpu.sync_copy(src_hbm.at[pl.ds(off, b)], sidx)
            pltpu.sync_copy(ones_v, acc.at[sidx], add=True)
            return 0
        lax.fori_loop(0, nchunk, chunk, 0)
        plsc.subcore_barrier()
        pltpu.sync_copy(acc.at[pl.ds(s * rt, rt)],
                        out_hbm.at[c, pl.ds(s * rt, rt)])

    return deg_kernel


@functools.lru_cache(maxsize=None)
def _make_prop(n, e, d):
    nw = _NC * _NS
    ew = e // nw
    b = 80
    assert ew % b == 0 and e % nw == 0 and d % _LANES == 0
    nchunk = ew // b
    npad = -(-n // 128) * 128
    rt = npad // _NS
    zr = 128
    mesh = plsc.VectorSubcoreMesh(core_axis_name="c", subcore_axis_name="s",
                                  num_cores=_NC, num_subcores=_NS)

    @functools.partial(
        pl.kernel,
        out_type=jax.ShapeDtypeStruct((_NC, npad, d), jnp.float32),
        mesh=mesh,
        scratch_types=[
            pltpu.VMEM((b,), jnp.int32),
            pltpu.VMEM((b,), jnp.int32),
            pltpu.VMEM((b, d), jnp.float32),
            pltpu.VMEM((zr, d), jnp.float32),
            pltpu.VMEM_SHARED((npad, d), jnp.float32),
            pltpu.SemaphoreType.DMA,
        ],
    )
    def prop_kernel(g_hbm, src_hbm, dst_hbm, out_hbm,
                    sidx, didx, rows, zbuf, acc, sem):
        c = lax.axis_index("c")
        s = lax.axis_index("s")
        wid = s * _NC + c

        def zfill(i, _):
            for j in range(d // _LANES):
                zbuf[i, pl.ds(j * _LANES, _LANES)] = (
                    jnp.zeros((_LANES,), jnp.float32))
            return 0
        lax.fori_loop(0, zr, zfill, 0)
        for r in range(rt // zr):
            pltpu.sync_copy(zbuf, acc.at[pl.ds(s * rt + r * zr, zr)])
        if rt % zr:
            pltpu.sync_copy(zbuf.at[pl.ds(0, rt % zr)],
                            acc.at[pl.ds(s * rt + (rt // zr) * zr, rt % zr)])
        plsc.subcore_barrier()

        base = wid * ew

        def chunk(ci, _):
            off = pl.multiple_of(base + ci * b, 8)
            pltpu.sync_copy(src_hbm.at[pl.ds(off, b)], sidx)
            pltpu.sync_copy(dst_hbm.at[pl.ds(off, b)], didx)
            pltpu.async_copy(g_hbm.at[sidx], rows, sem).wait()
            pltpu.sync_copy(rows, acc.at[didx], add=True)
            return 0
        lax.fori_loop(0, nchunk, chunk, 0)
        plsc.subcore_barrier()
        pltpu.sync_copy(acc.at[pl.ds(s * rt, rt)],
                        out_hbm.at[c, pl.ds(s * rt, rt)])

    return prop_kernel


_R = 1000


def _prep_body(degp_ref, x_ref, dis_ref, g_ref):
    deg = (degp_ref[0] + degp_ref[1])[:, 0:1]
    dis = jnp.where(deg > 0, lax.rsqrt(deg), 0.0)
    dis_ref[...] = dis
    g_ref[...] = dis * x_ref[...]


@functools.lru_cache(maxsize=None)
def _make_prep(n, d, w):
    r = _R
    return pl.pallas_call(
        _prep_body,
        grid=(n // r,),
        in_specs=[pl.BlockSpec((_NC, r, w), lambda i: (0, i, 0)),
                  pl.BlockSpec((r, d), lambda i: (i, 0))],
        out_specs=[pl.BlockSpec((r, 1), lambda i: (i, 0)),
                   pl.BlockSpec((r, d), lambda i: (i, 0))],
        out_shape=[jax.ShapeDtypeStruct((n, 1), jnp.float32),
                   jax.ShapeDtypeStruct((n, d), jnp.float32)],
    )


def _combine_body(p_ref, dis_ref, t1_ref, g2_ref):
    dis = dis_ref[...]
    t1 = -dis * (p_ref[0] + p_ref[1])
    t1_ref[...] = t1
    g2_ref[...] = dis * t1


@functools.lru_cache(maxsize=None)
def _make_combine(n, d):
    r = _R
    return pl.pallas_call(
        _combine_body,
        grid=(n // r,),
        in_specs=[pl.BlockSpec((_NC, r, d), lambda i: (0, i, 0)),
                  pl.BlockSpec((r, 1), lambda i: (i, 0))],
        out_specs=[pl.BlockSpec((r, d), lambda i: (i, 0)),
                   pl.BlockSpec((r, d), lambda i: (i, 0))],
        out_shape=[jax.ShapeDtypeStruct((n, d), jnp.float32),
                   jax.ShapeDtypeStruct((n, d), jnp.float32)],
    )


def _layer_body(h_ref, t1_ref, q_ref, dis_ref, a_ref, bw_ref, cw_ref,
                bias_ref, hn_ref, gn_ref):
    dis = dis_ref[...]
    u = dis * (q_ref[0] + q_ref[1])
    z = (jnp.dot(h_ref[...], a_ref[...], preferred_element_type=jnp.float32)
         + jnp.dot(t1_ref[...], bw_ref[...],
                   preferred_element_type=jnp.float32)
         + jnp.dot(u, cw_ref[...], preferred_element_type=jnp.float32)
         + bias_ref[...])
    hn = jnp.maximum(z, 0.0)
    hn_ref[...] = hn
    gn_ref[...] = dis * hn


def _final_body(h_ref, t1_ref, q_ref, dis_ref, a_ref, bw_ref, cw_ref,
                bias_ref, out_ref):
    dis = dis_ref[...]
    u = dis * (q_ref[0] + q_ref[1])
    z = (jnp.dot(h_ref[...], a_ref[...], preferred_element_type=jnp.float32)
         + jnp.dot(t1_ref[...], bw_ref[...],
                   preferred_element_type=jnp.float32)
         + jnp.dot(u, cw_ref[...], preferred_element_type=jnp.float32)
         + bias_ref[...])
    out_ref[...] = jnp.tanh(z)


@functools.lru_cache(maxsize=None)
def _make_layer(n, d, dout, final):
    r = _R
    in_specs = [pl.BlockSpec((r, d), lambda i: (i, 0)),
                pl.BlockSpec((r, d), lambda i: (i, 0)),
                pl.BlockSpec((_NC, r, d), lambda i: (0, i, 0)),
                pl.BlockSpec((r, 1), lambda i: (i, 0)),
                pl.BlockSpec((d, dout), lambda i: (0, 0)),
                pl.BlockSpec((d, dout), lambda i: (0, 0)),
                pl.BlockSpec((d, dout), lambda i: (0, 0)),
                pl.BlockSpec((1, dout), lambda i: (0, 0))]
    if final:
        return pl.pallas_call(
            _final_body,
            grid=(n // r,),
            in_specs=in_specs,
            out_specs=pl.BlockSpec((r, dout), lambda i: (i, 0)),
            out_shape=jax.ShapeDtypeStruct((n, dout), jnp.float32),
        )
    return pl.pallas_call(
        _layer_body,
        grid=(n // r,),
        in_specs=in_specs,
        out_specs=[pl.BlockSpec((r, dout), lambda i: (i, 0)),
                   pl.BlockSpec((r, dout), lambda i: (i, 0))],
        out_shape=[jax.ShapeDtypeStruct((n, dout), jnp.float32),
                   jax.ShapeDtypeStruct((n, dout), jnp.float32)],
    )


def kernel(x, edge_index, W_in, b_in, W_h, b_h, W_out, b_out):
    n, d = x.shape
    e = edge_index.shape[1]
    src = edge_index[0]
    dst = edge_index[1]

    ones_c = jnp.ones((80, 128), jnp.float32)
    zeros_c = jnp.zeros((128, 128), jnp.float32)
    degp = _make_deg(n, e)(src, ones_c, zeros_c)
    dis, g = _make_prep(n, d, 128)(degp, x)

    prop = _make_prop(n, e, d)
    combine = _make_combine(n, d)

    h = x
    for W, bias, final in ((W_in, b_in, False), (W_h, b_h, False),
                           (W_out, b_out, True)):
        dout = W.shape[2]
        a_w = W[0] - W[2]
        b_w = W[1]
        c_w = -2.0 * W[2]
        p = prop(g, src, dst)
        t1, g2 = combine(p, dis)
        q = prop(g2, src, dst)
        layer = _make_layer(n, d, dout, final)
        res = layer(h, t1, q, dis, a_w, b_w, c_w, bias.reshape(1, dout))
        if final:
            return res
        h, g = res


# speedup vs baseline: 14.6146x; 2.2968x over previous
import functools

import jax
import jax.numpy as jnp
from jax import lax
from jax.experimental import pallas as pl
from jax.experimental.pallas import tpu as pltpu
from jax.experimental.pallas import tpu_sc as plsc

_NC = 2
_NS = 16
_LANES = 16


@functools.lru_cache(maxsize=None)
def _make_deg(n, e):
    nw = _NC * _NS
    ew = e // nw
    b = 80
    assert ew % b == 0 and e % nw == 0
    nchunk = ew // b
    npad = -(-n // 128) * 128
    rt = npad // _NS
    w = 128
    mesh = plsc.VectorSubcoreMesh(core_axis_name="c", subcore_axis_name="s",
                                  num_cores=_NC, num_subcores=_NS)

    @functools.partial(
        pl.kernel,
        out_type=jax.ShapeDtypeStruct((_NC, npad, w), jnp.float32),
        mesh=mesh,
        scratch_types=[
            pltpu.VMEM((nchunk, b), jnp.int32),
            pltpu.VMEM((b, w), jnp.float32),
            pltpu.VMEM((128, w), jnp.float32),
            pltpu.VMEM_SHARED((npad, w), jnp.float32),
        ],
    )
    def deg_kernel(src3, ones_hbm, zeros_hbm, out_hbm, srcv, ones_v, zbuf,
                   acc):
        c = lax.axis_index("c")
        s = lax.axis_index("s")
        wid = s * _NC + c

        pltpu.sync_copy(zeros_hbm, zbuf)
        for r in range(rt // 128):
            pltpu.sync_copy(zbuf, acc.at[pl.ds(s * rt + r * 128, 128)])
        if rt % 128:
            pltpu.sync_copy(zbuf.at[pl.ds(0, rt % 128)],
                            acc.at[pl.ds(s * rt + (rt // 128) * 128,
                                         rt % 128)])
        pltpu.sync_copy(src3.at[wid], srcv)
        pltpu.sync_copy(ones_hbm, ones_v)
        plsc.subcore_barrier()

        def chunk(ci, _):
            pltpu.sync_copy(ones_v, acc.at[srcv.at[ci]], add=True)
            return 0
        lax.fori_loop(0, nchunk, chunk, 0)
        plsc.subcore_barrier()
        pltpu.sync_copy(acc.at[pl.ds(s * rt, rt)],
                        out_hbm.at[c, pl.ds(s * rt, rt)])

    return deg_kernel


@functools.lru_cache(maxsize=None)
def _make_prop(n, e, d):
    nw = _NC * _NS
    ew = e // nw
    b = 40
    nbuf = 5
    assert ew % (b * nbuf) == 0 and e % nw == 0 and d % _LANES == 0
    ngrp = ew // (b * nbuf)
    assert ngrp % 2 == 0
    npad = -(-n // 128) * 128
    rt = npad // _NS
    zr = 64
    mesh = plsc.VectorSubcoreMesh(core_axis_name="c", subcore_axis_name="s",
                                  num_cores=_NC, num_subcores=_NS)

    @functools.partial(
        pl.kernel,
        out_type=jax.ShapeDtypeStruct((_NC, npad, d), jnp.float32),
        mesh=mesh,
        scratch_types=(
            [pltpu.VMEM((nbuf, b), jnp.int32) for _ in range(4)]
            + [pltpu.VMEM((b, d), jnp.float32) for _ in range(nbuf)]
            + [pltpu.SemaphoreType.DMA for _ in range(nbuf + 2)]
            + [pltpu.VMEM((zr, d), jnp.float32),
               pltpu.VMEM_SHARED((npad, d), jnp.float32)]
        ),
    )
    def prop_kernel(g_hbm, src4, dst4, zeros_hbm, out_hbm, *rest):
        sgrp = rest[0:2]
        dgrp = rest[2:4]
        bufs = rest[4:4 + nbuf]
        sems = rest[4 + nbuf:4 + 2 * nbuf]
        isems = rest[4 + 2 * nbuf:6 + 2 * nbuf]
        zbuf = rest[6 + 2 * nbuf]
        acc = rest[7 + 2 * nbuf]
        c = lax.axis_index("c")
        s = lax.axis_index("s")
        wid = s * _NC + c

        pltpu.sync_copy(zeros_hbm, zbuf)
        for r in range(rt // zr):
            pltpu.sync_copy(zbuf, acc.at[pl.ds(s * rt + r * zr, zr)])
        if rt % zr:
            pltpu.sync_copy(zbuf.at[pl.ds(0, rt % zr)],
                            acc.at[pl.ds(s * rt + (rt // zr) * zr, rt % zr)])
        plsc.subcore_barrier()

        def fetch_idx(t, st):
            pltpu.async_copy(src4.at[wid, t], sgrp[st], isems[st])
            pltpu.async_copy(dst4.at[wid, t], dgrp[st], isems[st])

        def wait_idx(t, st):
            pltpu.make_async_copy(src4.at[wid, t], sgrp[st],
                                  isems[st]).wait()
            pltpu.make_async_copy(dst4.at[wid, t], dgrp[st],
                                  isems[st]).wait()

        def start_g(st, j):
            pltpu.async_copy(g_hbm.at[sgrp[st].at[j]], bufs[j], sems[j])

        def wait_g(st, j):
            pltpu.make_async_copy(g_hbm.at[sgrp[st].at[j]], bufs[j],
                                  sems[j]).wait()

        fetch_idx(0, 0)
        wait_idx(0, 0)
        for j in range(nbuf):
            start_g(0, j)
        fetch_idx(1, 1)

        def pair(u, _):
            for t2 in range(2):
                t = u * 2 + t2
                st = t2
                ost = 1 - t2

                @pl.when(t + 1 < ngrp)
                def _():
                    wait_idx(t + 1, ost)
                for j in range(nbuf):
                    wait_g(st, j)
                    pltpu.sync_copy(bufs[j], acc.at[dgrp[st].at[j]],
                                    add=True)

                    @pl.when(t + 1 < ngrp)
                    def _():
                        start_g(ost, j)

                @pl.when(t + 2 < ngrp)
                def _():
                    fetch_idx(t + 2, st)
            return 0
        lax.fori_loop(0, ngrp // 2, pair, 0)
        plsc.subcore_barrier()
        pltpu.sync_copy(acc.at[pl.ds(s * rt, rt)],
                        out_hbm.at[c, pl.ds(s * rt, rt)])

    return prop_kernel


_R = 1000


def _prep_body(degp_ref, x_ref, dis_ref, g_ref):
    deg = (degp_ref[0] + degp_ref[1])[:, 0:1]
    dis = jnp.where(deg > 0, lax.rsqrt(deg), 0.0)
    dis_ref[...] = dis
    g_ref[...] = dis * x_ref[...]


@functools.lru_cache(maxsize=None)
def _make_prep(n, d, w):
    r = _R
    return pl.pallas_call(
        _prep_body,
        grid=(n // r,),
        in_specs=[pl.BlockSpec((_NC, r, w), lambda i: (0, i, 0)),
                  pl.BlockSpec((r, d), lambda i: (i, 0))],
        out_specs=[pl.BlockSpec((r, 1), lambda i: (i, 0)),
                   pl.BlockSpec((r, d), lambda i: (i, 0))],
        out_shape=[jax.ShapeDtypeStruct((n, 1), jnp.float32),
                   jax.ShapeDtypeStruct((n, d), jnp.float32)],
    )


def _combine_body(p_ref, dis_ref, t1_ref, g2_ref):
    dis = dis_ref[...]
    t1 = -dis * (p_ref[0] + p_ref[1])
    t1_ref[...] = t1
    g2_ref[...] = dis * t1


@functools.lru_cache(maxsize=None)
def _make_combine(n, d):
    r = _R
    return pl.pallas_call(
        _combine_body,
        grid=(n // r,),
        in_specs=[pl.BlockSpec((_NC, r, d), lambda i: (0, i, 0)),
                  pl.BlockSpec((r, 1), lambda i: (i, 0))],
        out_specs=[pl.BlockSpec((r, d), lambda i: (i, 0)),
                   pl.BlockSpec((r, d), lambda i: (i, 0))],
        out_shape=[jax.ShapeDtypeStruct((n, d), jnp.float32),
                   jax.ShapeDtypeStruct((n, d), jnp.float32)],
    )


def _layer_body(h_ref, t1_ref, q_ref, dis_ref, a_ref, bw_ref, cw_ref,
                bias_ref, hn_ref, gn_ref):
    dis = dis_ref[...]
    u = dis * (q_ref[0] + q_ref[1])
    z = (jnp.dot(h_ref[...], a_ref[...], preferred_element_type=jnp.float32)
         + jnp.dot(t1_ref[...], bw_ref[...],
                   preferred_element_type=jnp.float32)
         + jnp.dot(u, cw_ref[...], preferred_element_type=jnp.float32)
         + bias_ref[...])
    hn = jnp.maximum(z, 0.0)
    hn_ref[...] = hn
    gn_ref[...] = dis * hn


def _final_body(h_ref, t1_ref, q_ref, dis_ref, a_ref, bw_ref, cw_ref,
                bias_ref, out_ref):
    dis = dis_ref[...]
    u = dis * (q_ref[0] + q_ref[1])
    z = (jnp.dot(h_ref[...], a_ref[...], preferred_element_type=jnp.float32)
         + jnp.dot(t1_ref[...], bw_ref[...],
                   preferred_element_type=jnp.float32)
         + jnp.dot(u, cw_ref[...], preferred_element_type=jnp.float32)
         + bias_ref[...])
    out_ref[...] = jnp.tanh(z)


@functools.lru_cache(maxsize=None)
def _make_layer(n, d, dout, final):
    r = _R
    in_specs = [pl.BlockSpec((r, d), lambda i: (i, 0)),
                pl.BlockSpec((r, d), lambda i: (i, 0)),
                pl.BlockSpec((_NC, r, d), lambda i: (0, i, 0)),
                pl.BlockSpec((r, 1), lambda i: (i, 0)),
                pl.BlockSpec((d, dout), lambda i: (0, 0)),
                pl.BlockSpec((d, dout), lambda i: (0, 0)),
                pl.BlockSpec((d, dout), lambda i: (0, 0)),
                pl.BlockSpec((1, dout), lambda i: (0, 0))]
    if final:
        return pl.pallas_call(
            _final_body,
            grid=(n // r,),
            in_specs=in_specs,
            out_specs=pl.BlockSpec((r, dout), lambda i: (i, 0)),
            out_shape=jax.ShapeDtypeStruct((n, dout), jnp.float32),
        )
    return pl.pallas_call(
        _layer_body,
        grid=(n // r,),
        in_specs=in_specs,
        out_specs=[pl.BlockSpec((r, dout), lambda i: (i, 0)),
                   pl.BlockSpec((r, dout), lambda i: (i, 0))],
        out_shape=[jax.ShapeDtypeStruct((n, dout), jnp.float32),
                   jax.ShapeDtypeStruct((n, dout), jnp.float32)],
    )


def kernel(x, edge_index, W_in, b_in, W_h, b_h, W_out, b_out):
    n, d = x.shape
    e = edge_index.shape[1]
    nw = _NC * _NS
    npad = -(-n // 128) * 128
    rt = npad // _NS
    src3 = edge_index[0].reshape(nw, e // nw // 80, 80)
    src4 = edge_index[0].reshape(nw, e // nw // 200, 5, 40)
    dst4 = edge_index[1].reshape(nw, e // nw // 200, 5, 40)

    ones_c = jnp.ones((80, 128), jnp.float32)
    zeros_c = jnp.zeros((128, 128), jnp.float32)
    zeros64_c = jnp.zeros((64, 128), jnp.float32)
    degp = _make_deg(n, e)(src3, ones_c, zeros_c)
    dis, g = _make_prep(n, d, 128)(degp, x)

    prop = _make_prop(n, e, d)
    combine = _make_combine(n, d)

    h = x
    for W, bias, final in ((W_in, b_in, False), (W_h, b_h, False),
                           (W_out, b_out, True)):
        dout = W.shape[2]
        a_w = W[0] - W[2]
        b_w = W[1]
        c_w = -2.0 * W[2]
        p = prop(g, src4, dst4, zeros64_c)
        t1, g2 = combine(p, dis)
        q = prop(g2, src4, dst4, zeros64_c)
        layer = _make_layer(n, d, dout, final)
        res = layer(h, t1, q, dis, a_w, b_w, c_w, bias.reshape(1, dout))
        if final:
            return res
        h, g = res


# speedup vs baseline: 14.8006x; 1.0127x over previous
import functools

import jax
import jax.numpy as jnp
from jax import lax
from jax.experimental import pallas as pl
from jax.experimental.pallas import tpu as pltpu
from jax.experimental.pallas import tpu_sc as plsc

_NC = 2
_NS = 16
_LANES = 16


@functools.lru_cache(maxsize=None)
def _make_deg(n, e):
    nw = _NC * _NS
    ew = e // nw
    b = 80
    assert ew % b == 0 and e % nw == 0
    nchunk = ew // b
    npad = -(-n // 128) * 128
    rt = npad // _NS
    w = 128
    mesh = plsc.VectorSubcoreMesh(core_axis_name="c", subcore_axis_name="s",
                                  num_cores=_NC, num_subcores=_NS)

    @functools.partial(
        pl.kernel,
        out_type=jax.ShapeDtypeStruct((_NC, npad, w), jnp.float32),
        mesh=mesh,
        scratch_types=[
            pltpu.VMEM((nchunk, b), jnp.int32),
            pltpu.VMEM((b, w), jnp.float32),
            pltpu.VMEM((128, w), jnp.float32),
            pltpu.VMEM_SHARED((npad, w), jnp.float32),
            pltpu.SemaphoreType.DMA,
        ],
    )
    def deg_kernel(src3, ones_hbm, zeros_hbm, out_hbm, srcv, ones_v, zbuf,
                   acc, ssem):
        c = lax.axis_index("c")
        s = lax.axis_index("s")
        wid = s * _NC + c

        pltpu.sync_copy(zeros_hbm, zbuf)
        for r in range(rt // 128):
            pltpu.sync_copy(zbuf, acc.at[pl.ds(s * rt + r * 128, 128)])
        if rt % 128:
            pltpu.sync_copy(zbuf.at[pl.ds(0, rt % 128)],
                            acc.at[pl.ds(s * rt + (rt // 128) * 128,
                                         rt % 128)])
        pltpu.sync_copy(src3.at[wid], srcv)
        pltpu.sync_copy(ones_hbm, ones_v)
        plsc.subcore_barrier()

        def chunk(ci, _):
            pltpu.async_copy(ones_v, acc.at[srcv.at[ci]], ssem, add=True)

            @pl.when(ci >= 8)
            def _():
                pltpu.make_async_copy(ones_v, acc.at[srcv.at[0]],
                                      ssem).wait()
            return 0
        lax.fori_loop(0, nchunk, chunk, 0)
        for _ in range(8):
            pltpu.make_async_copy(ones_v, acc.at[srcv.at[0]], ssem).wait()
        plsc.subcore_barrier()
        pltpu.sync_copy(acc.at[pl.ds(s * rt, rt)],
                        out_hbm.at[c, pl.ds(s * rt, rt)])

    return deg_kernel


@functools.lru_cache(maxsize=None)
def _make_prop(n, e, d):
    nw = _NC * _NS
    ew = e // nw
    b = 40
    nbuf = 5
    assert ew % (b * nbuf) == 0 and e % nw == 0 and d % _LANES == 0
    ngrp = ew // (b * nbuf)
    assert ngrp % 2 == 0
    npad = -(-n // 128) * 128
    rt = npad // _NS
    zr = 64
    mesh = plsc.VectorSubcoreMesh(core_axis_name="c", subcore_axis_name="s",
                                  num_cores=_NC, num_subcores=_NS)

    @functools.partial(
        pl.kernel,
        out_type=jax.ShapeDtypeStruct((_NC, npad, d), jnp.float32),
        mesh=mesh,
        scratch_types=(
            [pltpu.VMEM((nbuf, b), jnp.int32) for _ in range(4)]
            + [pltpu.VMEM((b, d), jnp.float32) for _ in range(nbuf)]
            + [pltpu.SemaphoreType.DMA for _ in range(2 * nbuf + 2)]
            + [pltpu.VMEM((zr, d), jnp.float32),
               pltpu.VMEM_SHARED((npad, d), jnp.float32)]
        ),
    )
    def prop_kernel(g_hbm, src4, dst4, zeros_hbm, out_hbm, *rest):
        sgrp = rest[0:2]
        dgrp = rest[2:4]
        bufs = rest[4:4 + nbuf]
        sems = rest[4 + nbuf:4 + 2 * nbuf]
        ssems = rest[4 + 2 * nbuf:4 + 3 * nbuf]
        isems = rest[4 + 3 * nbuf:6 + 3 * nbuf]
        zbuf = rest[6 + 3 * nbuf]
        acc = rest[7 + 3 * nbuf]
        c = lax.axis_index("c")
        s = lax.axis_index("s")
        wid = s * _NC + c

        pltpu.sync_copy(zeros_hbm, zbuf)
        for r in range(rt // zr):
            pltpu.sync_copy(zbuf, acc.at[pl.ds(s * rt + r * zr, zr)])
        if rt % zr:
            pltpu.sync_copy(zbuf.at[pl.ds(0, rt % zr)],
                            acc.at[pl.ds(s * rt + (rt // zr) * zr, rt % zr)])
        plsc.subcore_barrier()

        def fetch_idx(t, st):
            pltpu.async_copy(src4.at[wid, t], sgrp[st], isems[st])
            pltpu.async_copy(dst4.at[wid, t], dgrp[st], isems[st])

        def wait_idx(t, st):
            pltpu.make_async_copy(src4.at[wid, t], sgrp[st],
                                  isems[st]).wait()
            pltpu.make_async_copy(dst4.at[wid, t], dgrp[st],
                                  isems[st]).wait()

        def start_g(st, j):
            pltpu.async_copy(g_hbm.at[sgrp[st].at[j]], bufs[j], sems[j])

        def wait_g(st, j):
            pltpu.make_async_copy(g_hbm.at[sgrp[st].at[j]], bufs[j],
                                  sems[j]).wait()

        fetch_idx(0, 0)
        wait_idx(0, 0)
        for j in range(nbuf):
            start_g(0, j)
        fetch_idx(1, 1)

        def start_s(st, j):
            pltpu.async_copy(bufs[j], acc.at[dgrp[st].at[j]], ssems[j],
                             add=True)

        def wait_s(st, j):
            pltpu.make_async_copy(bufs[j], acc.at[dgrp[st].at[j]],
                                  ssems[j]).wait()

        def pair(u, _):
            for t2 in range(2):
                t = u * 2 + t2
                st = t2
                ost = 1 - t2

                for j in range(nbuf):
                    wait_g(st, j)
                    start_s(st, j)

                @pl.when(t + 1 < ngrp)
                def _():
                    wait_idx(t + 1, ost)
                for j in range(nbuf):
                    wait_s(st, j)

                    @pl.when(t + 1 < ngrp)
                    def _():
                        start_g(ost, j)

                @pl.when(t + 2 < ngrp)
                def _():
                    fetch_idx(t + 2, st)
            return 0
        lax.fori_loop(0, ngrp // 2, pair, 0)
        plsc.subcore_barrier()
        pltpu.sync_copy(acc.at[pl.ds(s * rt, rt)],
                        out_hbm.at[c, pl.ds(s * rt, rt)])

    return prop_kernel


_R = 1000


def _prep_body(degp_ref, x_ref, dis_ref, g_ref):
    deg = (degp_ref[0] + degp_ref[1])[:, 0:1]
    dis = jnp.where(deg > 0, lax.rsqrt(deg), 0.0)
    dis_ref[...] = dis
    g_ref[...] = dis * x_ref[...]


@functools.lru_cache(maxsize=None)
def _make_prep(n, d, w):
    r = _R
    return pl.pallas_call(
        _prep_body,
        grid=(n // r,),
        in_specs=[pl.BlockSpec((_NC, r, w), lambda i: (0, i, 0)),
                  pl.BlockSpec((r, d), lambda i: (i, 0))],
        out_specs=[pl.BlockSpec((r, 1), lambda i: (i, 0)),
                   pl.BlockSpec((r, d), lambda i: (i, 0))],
        out_shape=[jax.ShapeDtypeStruct((n, 1), jnp.float32),
                   jax.ShapeDtypeStruct((n, d), jnp.float32)],
    )


def _combine_body(p_ref, dis_ref, t1_ref, g2_ref):
    dis = dis_ref[...]
    t1 = -dis * (p_ref[0] + p_ref[1])
    t1_ref[...] = t1
    g2_ref[...] = dis * t1


@functools.lru_cache(maxsize=None)
def _make_combine(n, d):
    r = _R
    return pl.pallas_call(
        _combine_body,
        grid=(n // r,),
        in_specs=[pl.BlockSpec((_NC, r, d), lambda i: (0, i, 0)),
                  pl.BlockSpec((r, 1), lambda i: (i, 0))],
        out_specs=[pl.BlockSpec((r, d), lambda i: (i, 0)),
                   pl.BlockSpec((r, d), lambda i: (i, 0))],
        out_shape=[jax.ShapeDtypeStruct((n, d), jnp.float32),
                   jax.ShapeDtypeStruct((n, d), jnp.float32)],
    )


def _layer_body(h_ref, t1_ref, q_ref, dis_ref, a_ref, bw_ref, cw_ref,
                bias_ref, hn_ref, gn_ref):
    dis = dis_ref[...]
    u = dis * (q_ref[0] + q_ref[1])
    z = (jnp.dot(h_ref[...], a_ref[...], preferred_element_type=jnp.float32)
         + jnp.dot(t1_ref[...], bw_ref[...],
                   preferred_element_type=jnp.float32)
         + jnp.dot(u, cw_ref[...], preferred_element_type=jnp.float32)
         + bias_ref[...])
    hn = jnp.maximum(z, 0.0)
    hn_ref[...] = hn
    gn_ref[...] = dis * hn


def _final_body(h_ref, t1_ref, q_ref, dis_ref, a_ref, bw_ref, cw_ref,
                bias_ref, out_ref):
    dis = dis_ref[...]
    u = dis * (q_ref[0] + q_ref[1])
    z = (jnp.dot(h_ref[...], a_ref[...], preferred_element_type=jnp.float32)
         + jnp.dot(t1_ref[...], bw_ref[...],
                   preferred_element_type=jnp.float32)
         + jnp.dot(u, cw_ref[...], preferred_element_type=jnp.float32)
         + bias_ref[...])
    out_ref[...] = jnp.tanh(z)


@functools.lru_cache(maxsize=None)
def _make_layer(n, d, dout, final):
    r = _R
    in_specs = [pl.BlockSpec((r, d), lambda i: (i, 0)),
                pl.BlockSpec((r, d), lambda i: (i, 0)),
                pl.BlockSpec((_NC, r, d), lambda i: (0, i, 0)),
                pl.BlockSpec((r, 1), lambda i: (i, 0)),
                pl.BlockSpec((d, dout), lambda i: (0, 0)),
                pl.BlockSpec((d, dout), lambda i: (0, 0)),
                pl.BlockSpec((d, dout), lambda i: (0, 0)),
                pl.BlockSpec((1, dout), lambda i: (0, 0))]
    if final:
        return pl.pallas_call(
            _final_body,
            grid=(n // r,),
            in_specs=in_specs,
            out_specs=pl.BlockSpec((r, dout), lambda i: (i, 0)),
            out_shape=jax.ShapeDtypeStruct((n, dout), jnp.float32),
        )
    return pl.pallas_call(
        _layer_body,
        grid=(n // r,),
        in_specs=in_specs,
        out_specs=[pl.BlockSpec((r, dout), lambda i: (i, 0)),
                   pl.BlockSpec((r, dout), lambda i: (i, 0))],
        out_shape=[jax.ShapeDtypeStruct((n, dout), jnp.float32),
                   jax.ShapeDtypeStruct((n, dout), jnp.float32)],
    )


def kernel(x, edge_index, W_in, b_in, W_h, b_h, W_out, b_out):
    n, d = x.shape
    e = edge_index.shape[1]
    nw = _NC * _NS
    npad = -(-n // 128) * 128
    rt = npad // _NS
    src3 = edge_index[0].reshape(nw, e // nw // 80, 80)
    src4 = edge_index[0].reshape(nw, e // nw // 200, 5, 40)
    dst4 = edge_index[1].reshape(nw, e // nw // 200, 5, 40)

    ones_c = jnp.ones((80, 128), jnp.float32)
    zeros_c = jnp.zeros((128, 128), jnp.float32)
    zeros64_c = jnp.zeros((64, 128), jnp.float32)
    degp = _make_deg(n, e)(src3, ones_c, zeros_c)
    dis, g = _make_prep(n, d, 128)(degp, x)

    prop = _make_prop(n, e, d)
    combine = _make_combine(n, d)

    h = x
    for W, bias, final in ((W_in, b_in, False), (W_h, b_h, False),
                           (W_out, b_out, True)):
        dout = W.shape[2]
        a_w = W[0] - W[2]
        b_w = W[1]
        c_w = -2.0 * W[2]
        p = prop(g, src4, dst4, zeros64_c)
        t1, g2 = combine(p, dis)
        q = prop(g2, src4, dst4, zeros64_c)
        layer = _make_layer(n, d, dout, final)
        res = layer(h, t1, q, dis, a_w, b_w, c_w, bias.reshape(1, dout))
        if final:
            return res
        h, g = res


# speedup vs baseline: 15.6653x; 1.0584x over previous
import functools

import jax
import jax.numpy as jnp
from jax import lax
from jax.experimental import pallas as pl
from jax.experimental.pallas import tpu as pltpu
from jax.experimental.pallas import tpu_sc as plsc

_NC = 2
_NS = 16
_LANES = 16


@functools.lru_cache(maxsize=None)
def _make_deg(n, e):
    nw = _NC * _NS
    ew = e // nw
    b = 80
    assert ew % b == 0 and e % nw == 0
    nchunk = ew // b
    npad = -(-n // 128) * 128
    rt1 = npad // _NS
    mesh = plsc.VectorSubcoreMesh(core_axis_name="c", subcore_axis_name="s",
                                  num_cores=_NC, num_subcores=_NS)

    @functools.partial(
        pl.kernel,
        out_type=jax.ShapeDtypeStruct((_NC * npad,), jnp.float32),
        mesh=mesh,
        scratch_types=[
            pltpu.VMEM((nchunk, b), jnp.int32),
            pltpu.VMEM((b,), jnp.float32),
            pltpu.VMEM_SHARED((npad,), jnp.float32),
            pltpu.SemaphoreType.DMA,
        ],
    )
    def deg_kernel(src3, ones_hbm, zeros_hbm, out_hbm, srcv, onesv, acc,
                   ssem):
        c = lax.axis_index("c")
        s = lax.axis_index("s")
        wid = s * _NC + c

        pltpu.sync_copy(ones_hbm, onesv)
        pltpu.sync_copy(src3.at[wid], srcv)

        @pl.when(s == 0)
        def _():
            pltpu.sync_copy(zeros_hbm, acc)
        plsc.subcore_barrier()

        def chunk(ci, _):
            pltpu.async_copy(onesv, acc.at[srcv.at[ci]], ssem, add=True)

            @pl.when(ci >= 8)
            def _():
                pltpu.make_async_copy(onesv, acc.at[srcv.at[0]],
                                      ssem).wait()
            return 0
        lax.fori_loop(0, nchunk, chunk, 0)
        for _ in range(8):
            pltpu.make_async_copy(onesv, acc.at[srcv.at[0]], ssem).wait()
        plsc.subcore_barrier()

        @pl.when(s == 0)
        def _():
            pltpu.sync_copy(acc,
                            out_hbm.at[pl.ds(pl.multiple_of(c * npad, 8),
                                             npad)])

    return deg_kernel


@functools.lru_cache(maxsize=None)
def _make_prop(n, e, d):
    nw = _NC * _NS
    ew = e // nw
    b = 40
    nbuf = 5
    assert ew % (b * nbuf) == 0 and e % nw == 0 and d % _LANES == 0
    ngrp = ew // (b * nbuf)
    assert ngrp % 2 == 0
    npad = -(-n // 128) * 128
    rt = npad // _NS
    zr = 64
    mesh = plsc.VectorSubcoreMesh(core_axis_name="c", subcore_axis_name="s",
                                  num_cores=_NC, num_subcores=_NS)

    @functools.partial(
        pl.kernel,
        out_type=jax.ShapeDtypeStruct((_NC, npad, d), jnp.float32),
        mesh=mesh,
        scratch_types=(
            [pltpu.VMEM((nbuf, b), jnp.int32) for _ in range(4)]
            + [pltpu.VMEM((b, d), jnp.float32) for _ in range(nbuf)]
            + [pltpu.SemaphoreType.DMA for _ in range(2 * nbuf + 2)]
            + [pltpu.VMEM((zr, d), jnp.float32),
               pltpu.VMEM_SHARED((npad, d), jnp.float32)]
        ),
    )
    def prop_kernel(g_hbm, src4, dst4, zeros_hbm, out_hbm, *rest):
        sgrp = rest[0:2]
        dgrp = rest[2:4]
        bufs = rest[4:4 + nbuf]
        sems = rest[4 + nbuf:4 + 2 * nbuf]
        ssems = rest[4 + 2 * nbuf:4 + 3 * nbuf]
        isems = rest[4 + 3 * nbuf:6 + 3 * nbuf]
        zbuf = rest[6 + 3 * nbuf]
        acc = rest[7 + 3 * nbuf]
        c = lax.axis_index("c")
        s = lax.axis_index("s")
        wid = s * _NC + c

        pltpu.sync_copy(zeros_hbm, zbuf)
        for r in range(rt // zr):
            pltpu.sync_copy(zbuf, acc.at[pl.ds(s * rt + r * zr, zr)])
        if rt % zr:
            pltpu.sync_copy(zbuf.at[pl.ds(0, rt % zr)],
                            acc.at[pl.ds(s * rt + (rt // zr) * zr, rt % zr)])
        plsc.subcore_barrier()

        def fetch_idx(t, st):
            pltpu.async_copy(src4.at[wid, t], sgrp[st], isems[st])
            pltpu.async_copy(dst4.at[wid, t], dgrp[st], isems[st])

        def wait_idx(t, st):
            pltpu.make_async_copy(src4.at[wid, t], sgrp[st],
                                  isems[st]).wait()
            pltpu.make_async_copy(dst4.at[wid, t], dgrp[st],
                                  isems[st]).wait()

        def start_g(st, j):
            pltpu.async_copy(g_hbm.at[sgrp[st].at[j]], bufs[j], sems[j])

        def wait_g(st, j):
            pltpu.make_async_copy(g_hbm.at[sgrp[st].at[j]], bufs[j],
                                  sems[j]).wait()

        fetch_idx(0, 0)
        wait_idx(0, 0)
        for j in range(nbuf):
            start_g(0, j)
        fetch_idx(1, 1)

        def start_s(st, j):
            pltpu.async_copy(bufs[j], acc.at[dgrp[st].at[j]], ssems[j],
                             add=True)

        def wait_s(st, j):
            pltpu.make_async_copy(bufs[j], acc.at[dgrp[st].at[j]],
                                  ssems[j]).wait()

        def pair(u, _):
            for t2 in range(2):
                t = u * 2 + t2
                st = t2
                ost = 1 - t2

                for j in range(nbuf):
                    wait_g(st, j)
                    start_s(st, j)

                @pl.when(t + 1 < ngrp)
                def _():
                    wait_idx(t + 1, ost)
                for j in range(nbuf):
                    wait_s(st, j)

                    @pl.when(t + 1 < ngrp)
                    def _():
                        start_g(ost, j)

                @pl.when(t + 2 < ngrp)
                def _():
                    fetch_idx(t + 2, st)
            return 0
        lax.fori_loop(0, ngrp // 2, pair, 0)
        plsc.subcore_barrier()
        pltpu.sync_copy(acc.at[pl.ds(s * rt, rt)],
                        out_hbm.at[c, pl.ds(s * rt, rt)])

    return prop_kernel


_R = 1000


def _prep_body(degp_ref, x_ref, dis_ref, g_ref):
    deg = (degp_ref[0] + degp_ref[1])[:, 0:1]
    dis = jnp.where(deg > 0, lax.rsqrt(deg), 0.0)
    dis_ref[...] = dis
    g_ref[...] = dis * x_ref[...]


@functools.lru_cache(maxsize=None)
def _make_prep(n, d, w):
    r = _R
    return pl.pallas_call(
        _prep_body,
        grid=(n // r,),
        in_specs=[pl.BlockSpec((_NC, r, w), lambda i: (0, i, 0)),
                  pl.BlockSpec((r, d), lambda i: (i, 0))],
        out_specs=[pl.BlockSpec((r, 1), lambda i: (i, 0)),
                   pl.BlockSpec((r, d), lambda i: (i, 0))],
        out_shape=[jax.ShapeDtypeStruct((n, 1), jnp.float32),
                   jax.ShapeDtypeStruct((n, d), jnp.float32)],
    )


def _combine_body(p_ref, dis_ref, t1_ref, g2_ref):
    dis = dis_ref[...]
    t1 = -dis * (p_ref[0] + p_ref[1])
    t1_ref[...] = t1
    g2_ref[...] = dis * t1


@functools.lru_cache(maxsize=None)
def _make_combine(n, d):
    r = _R
    return pl.pallas_call(
        _combine_body,
        grid=(n // r,),
        in_specs=[pl.BlockSpec((_NC, r, d), lambda i: (0, i, 0)),
                  pl.BlockSpec((r, 1), lambda i: (i, 0))],
        out_specs=[pl.BlockSpec((r, d), lambda i: (i, 0)),
                   pl.BlockSpec((r, d), lambda i: (i, 0))],
        out_shape=[jax.ShapeDtypeStruct((n, d), jnp.float32),
                   jax.ShapeDtypeStruct((n, d), jnp.float32)],
    )


def _layer_body(h_ref, t1_ref, q_ref, dis_ref, a_ref, bw_ref, cw_ref,
                bias_ref, hn_ref, gn_ref):
    dis = dis_ref[...]
    u = dis * (q_ref[0] + q_ref[1])
    z = (jnp.dot(h_ref[...], a_ref[...], preferred_element_type=jnp.float32)
         + jnp.dot(t1_ref[...], bw_ref[...],
                   preferred_element_type=jnp.float32)
         + jnp.dot(u, cw_ref[...], preferred_element_type=jnp.float32)
         + bias_ref[...])
    hn = jnp.maximum(z, 0.0)
    hn_ref[...] = hn
    gn_ref[...] = dis * hn


def _final_body(h_ref, t1_ref, q_ref, dis_ref, a_ref, bw_ref, cw_ref,
                bias_ref, out_ref):
    dis = dis_ref[...]
    u = dis * (q_ref[0] + q_ref[1])
    z = (jnp.dot(h_ref[...], a_ref[...], preferred_element_type=jnp.float32)
         + jnp.dot(t1_ref[...], bw_ref[...],
                   preferred_element_type=jnp.float32)
         + jnp.dot(u, cw_ref[...], preferred_element_type=jnp.float32)
         + bias_ref[...])
    out_ref[...] = jnp.tanh(z)


@functools.lru_cache(maxsize=None)
def _make_layer(n, d, dout, final):
    r = _R
    in_specs = [pl.BlockSpec((r, d), lambda i: (i, 0)),
                pl.BlockSpec((r, d), lambda i: (i, 0)),
                pl.BlockSpec((_NC, r, d), lambda i: (0, i, 0)),
                pl.BlockSpec((r, 1), lambda i: (i, 0)),
                pl.BlockSpec((d, dout), lambda i: (0, 0)),
                pl.BlockSpec((d, dout), lambda i: (0, 0)),
                pl.BlockSpec((d, dout), lambda i: (0, 0)),
                pl.BlockSpec((1, dout), lambda i: (0, 0))]
    if final:
        return pl.pallas_call(
            _final_body,
            grid=(n // r,),
            in_specs=in_specs,
            out_specs=pl.BlockSpec((r, dout), lambda i: (i, 0)),
            out_shape=jax.ShapeDtypeStruct((n, dout), jnp.float32),
        )
    return pl.pallas_call(
        _layer_body,
        grid=(n // r,),
        in_specs=in_specs,
        out_specs=[pl.BlockSpec((r, dout), lambda i: (i, 0)),
                   pl.BlockSpec((r, dout), lambda i: (i, 0))],
        out_shape=[jax.ShapeDtypeStruct((n, dout), jnp.float32),
                   jax.ShapeDtypeStruct((n, dout), jnp.float32)],
    )


def kernel(x, edge_index, W_in, b_in, W_h, b_h, W_out, b_out):
    n, d = x.shape
    e = edge_index.shape[1]
    nw = _NC * _NS
    npad = -(-n // 128) * 128
    rt = npad // _NS
    src3 = edge_index[0].reshape(nw, e // nw // 80, 80)
    src4 = edge_index[0].reshape(nw, e // nw // 200, 5, 40)
    dst4 = edge_index[1].reshape(nw, e // nw // 200, 5, 40)

    ones1_c = jnp.ones((80,), jnp.float32)
    zeros1_c = jnp.zeros((npad,), jnp.float32)
    zeros64_c = jnp.zeros((64, 128), jnp.float32)
    degp = _make_deg(n, e)(src3, ones1_c, zeros1_c).reshape(_NC, npad, 1)

    dis, g = _make_prep(n, d, 1)(degp, x)

    prop = _make_prop(n, e, d)
    combine = _make_combine(n, d)

    h = x
    for W, bias, final in ((W_in, b_in, False), (W_h, b_h, False),
                           (W_out, b_out, True)):
        dout = W.shape[2]
        a_w = W[0] - W[2]
        b_w = W[1]
        c_w = -2.0 * W[2]
        p = prop(g, src4, dst4, zeros64_c)
        t1, g2 = combine(p, dis)
        q = prop(g2, src4, dst4, zeros64_c)
        layer = _make_layer(n, d, dout, final)
        res = layer(h, t1, q, dis, a_w, b_w, c_w, bias.reshape(1, dout))
        if final:
            return res
        h, g = res


# speedup vs baseline: 15.8141x; 1.0095x over previous
import functools

import jax
import jax.numpy as jnp
from jax import lax
from jax.experimental import pallas as pl
from jax.experimental.pallas import tpu as pltpu
from jax.experimental.pallas import tpu_sc as plsc

_NC = 2
_NS = 16
_LANES = 16


@functools.lru_cache(maxsize=None)
def _make_deg(n, e):
    nw = _NC * _NS
    ew = e // nw
    b = 80
    assert ew % b == 0 and e % nw == 0
    nchunk = ew // b
    npad = -(-n // 128) * 128
    rt1 = npad // _NS
    mesh = plsc.VectorSubcoreMesh(core_axis_name="c", subcore_axis_name="s",
                                  num_cores=_NC, num_subcores=_NS)

    @functools.partial(
        pl.kernel,
        out_type=jax.ShapeDtypeStruct((_NC * npad,), jnp.float32),
        mesh=mesh,
        scratch_types=[
            pltpu.VMEM((nchunk, b), jnp.int32),
            pltpu.VMEM((b,), jnp.float32),
            pltpu.VMEM_SHARED((npad,), jnp.float32),
            pltpu.SemaphoreType.DMA,
        ],
    )
    def deg_kernel(src3, ones_hbm, zeros_hbm, out_hbm, srcv, onesv, acc,
                   ssem):
        c = lax.axis_index("c")
        s = lax.axis_index("s")
        wid = s * _NC + c

        pltpu.sync_copy(ones_hbm, onesv)
        pltpu.sync_copy(src3.at[wid], srcv)

        @pl.when(s == 0)
        def _():
            pltpu.sync_copy(zeros_hbm, acc)
        plsc.subcore_barrier()

        def chunk(ci, _):
            pltpu.async_copy(onesv, acc.at[srcv.at[ci]], ssem, add=True)

            @pl.when(ci >= 8)
            def _():
                pltpu.make_async_copy(onesv, acc.at[srcv.at[0]],
                                      ssem).wait()
            return 0
        lax.fori_loop(0, nchunk, chunk, 0)
        for _ in range(8):
            pltpu.make_async_copy(onesv, acc.at[srcv.at[0]], ssem).wait()
        plsc.subcore_barrier()

        @pl.when(s == 0)
        def _():
            pltpu.sync_copy(acc,
                            out_hbm.at[pl.ds(pl.multiple_of(c * npad, 8),
                                             npad)])

    return deg_kernel


@functools.lru_cache(maxsize=None)
def _make_prop(n, e, d):
    nw = _NC * _NS
    ew = e // nw
    b = 40
    nbuf = 5
    assert ew % (b * nbuf) == 0 and e % nw == 0 and d % _LANES == 0
    ngrp = ew // (b * nbuf)
    assert ngrp % 2 == 0
    npad = -(-n // 128) * 128
    rt = npad // _NS
    zr = 64
    mesh = plsc.VectorSubcoreMesh(core_axis_name="c", subcore_axis_name="s",
                                  num_cores=_NC, num_subcores=_NS)

    @functools.partial(
        pl.kernel,
        out_type=jax.ShapeDtypeStruct((_NC, npad, d), jnp.float32),
        mesh=mesh,
        scratch_types=(
            [pltpu.VMEM((nbuf, b), jnp.int32) for _ in range(4)]
            + [pltpu.VMEM((b, d), jnp.float32) for _ in range(nbuf)]
            + [pltpu.SemaphoreType.DMA for _ in range(2 * nbuf + 3)]
            + [pltpu.VMEM((zr, d), jnp.float32),
               pltpu.VMEM_SHARED((npad, d), jnp.float32)]
        ),
    )
    def prop_kernel(g_hbm, src4, dst4, zeros_hbm, out_hbm, *rest):
        sgrp = rest[0:2]
        dgrp = rest[2:4]
        bufs = rest[4:4 + nbuf]
        sems = rest[4 + nbuf:4 + 2 * nbuf]
        ssems = rest[4 + 2 * nbuf:4 + 3 * nbuf]
        isems = rest[4 + 3 * nbuf:6 + 3 * nbuf]
        zsem = rest[6 + 3 * nbuf]
        zbuf = rest[7 + 3 * nbuf]
        acc = rest[8 + 3 * nbuf]
        c = lax.axis_index("c")
        s = lax.axis_index("s")
        wid = s * _NC + c

        def fetch_idx(t, st):
            pltpu.async_copy(src4.at[wid, t], sgrp[st], isems[st])
            pltpu.async_copy(dst4.at[wid, t], dgrp[st], isems[st])

        def wait_idx(t, st):
            pltpu.make_async_copy(src4.at[wid, t], sgrp[st],
                                  isems[st]).wait()
            pltpu.make_async_copy(dst4.at[wid, t], dgrp[st],
                                  isems[st]).wait()

        def start_g(st, j):
            pltpu.async_copy(g_hbm.at[sgrp[st].at[j]], bufs[j], sems[j])

        def wait_g(st, j):
            pltpu.make_async_copy(g_hbm.at[sgrp[st].at[j]], bufs[j],
                                  sems[j]).wait()

        fetch_idx(0, 0)
        pltpu.sync_copy(zeros_hbm, zbuf)
        for r in range(rt // zr):
            pltpu.async_copy(zbuf, acc.at[pl.ds(s * rt + r * zr, zr)], zsem)
        if rt % zr:
            pltpu.async_copy(zbuf.at[pl.ds(0, rt % zr)],
                            acc.at[pl.ds(s * rt + (rt // zr) * zr, rt % zr)],
                            zsem)
        wait_idx(0, 0)
        for j in range(nbuf):
            start_g(0, j)
        fetch_idx(1, 1)
        for r in range(rt // zr):
            pltpu.make_async_copy(
                zbuf, acc.at[pl.ds(s * rt + r * zr, zr)], zsem).wait()
        if rt % zr:
            pltpu.make_async_copy(
                zbuf.at[pl.ds(0, rt % zr)],
                acc.at[pl.ds(s * rt + (rt // zr) * zr, rt % zr)],
                zsem).wait()
        plsc.subcore_barrier()

        def start_s(st, j):
            pltpu.async_copy(bufs[j], acc.at[dgrp[st].at[j]], ssems[j],
                             add=True)

        def wait_s(st, j):
            pltpu.make_async_copy(bufs[j], acc.at[dgrp[st].at[j]],
                                  ssems[j]).wait()

        def pair(u, _):
            for t2 in range(2):
                t = u * 2 + t2
                st = t2
                ost = 1 - t2

                for j in range(nbuf):
                    wait_g(st, j)
                    start_s(st, j)

                @pl.when(t + 1 < ngrp)
                def _():
                    wait_idx(t + 1, ost)
                for j in range(nbuf):
                    wait_s(st, j)

                    @pl.when(t + 1 < ngrp)
                    def _():
                        start_g(ost, j)

                @pl.when(t + 2 < ngrp)
                def _():
                    fetch_idx(t + 2, st)
            return 0
        lax.fori_loop(0, ngrp // 2, pair, 0)
        plsc.subcore_barrier()
        pltpu.sync_copy(acc.at[pl.ds(s * rt, rt)],
                        out_hbm.at[c, pl.ds(s * rt, rt)])

    return prop_kernel


_R = 1000


def _prep_body(degp_ref, x_ref, dis_ref, g_ref):
    deg = (degp_ref[0] + degp_ref[1])[:, 0:1]
    dis = jnp.where(deg > 0, lax.rsqrt(deg), 0.0)
    dis_ref[...] = dis
    g_ref[...] = dis * x_ref[...]


@functools.lru_cache(maxsize=None)
def _make_prep(n, d, w):
    r = _R
    return pl.pallas_call(
        _prep_body,
        grid=(n // r,),
        in_specs=[pl.BlockSpec((_NC, r, w), lambda i: (0, i, 0)),
                  pl.BlockSpec((r, d), lambda i: (i, 0))],
        out_specs=[pl.BlockSpec((r, 1), lambda i: (i, 0)),
                   pl.BlockSpec((r, d), lambda i: (i, 0))],
        out_shape=[jax.ShapeDtypeStruct((n, 1), jnp.float32),
                   jax.ShapeDtypeStruct((n, d), jnp.float32)],
    )


def _combine_body(p_ref, dis_ref, t1_ref, g2_ref):
    dis = dis_ref[...]
    t1 = -dis * (p_ref[0] + p_ref[1])
    t1_ref[...] = t1
    g2_ref[...] = dis * t1


@functools.lru_cache(maxsize=None)
def _make_combine(n, d):
    r = _R
    return pl.pallas_call(
        _combine_body,
        grid=(n // r,),
        in_specs=[pl.BlockSpec((_NC, r, d), lambda i: (0, i, 0)),
                  pl.BlockSpec((r, 1), lambda i: (i, 0))],
        out_specs=[pl.BlockSpec((r, d), lambda i: (i, 0)),
                   pl.BlockSpec((r, d), lambda i: (i, 0))],
        out_shape=[jax.ShapeDtypeStruct((n, d), jnp.float32),
                   jax.ShapeDtypeStruct((n, d), jnp.float32)],
    )


def _layer_body(h_ref, t1_ref, q_ref, dis_ref, a_ref, bw_ref, cw_ref,
                bias_ref, hn_ref, gn_ref):
    dis = dis_ref[...]
    u = dis * (q_ref[0] + q_ref[1])
    z = (jnp.dot(h_ref[...], a_ref[...], preferred_element_type=jnp.float32)
         + jnp.dot(t1_ref[...], bw_ref[...],
                   preferred_element_type=jnp.float32)
         + jnp.dot(u, cw_ref[...], preferred_element_type=jnp.float32)
         + bias_ref[...])
    hn = jnp.maximum(z, 0.0)
    hn_ref[...] = hn
    gn_ref[...] = dis * hn


def _final_body(h_ref, t1_ref, q_ref, dis_ref, a_ref, bw_ref, cw_ref,
                bias_ref, out_ref):
    dis = dis_ref[...]
    u = dis * (q_ref[0] + q_ref[1])
    z = (jnp.dot(h_ref[...], a_ref[...], preferred_element_type=jnp.float32)
         + jnp.dot(t1_ref[...], bw_ref[...],
                   preferred_element_type=jnp.float32)
         + jnp.dot(u, cw_ref[...], preferred_element_type=jnp.float32)
         + bias_ref[...])
    out_ref[...] = jnp.tanh(z)


@functools.lru_cache(maxsize=None)
def _make_layer(n, d, dout, final):
    r = _R
    in_specs = [pl.BlockSpec((r, d), lambda i: (i, 0)),
                pl.BlockSpec((r, d), lambda i: (i, 0)),
                pl.BlockSpec((_NC, r, d), lambda i: (0, i, 0)),
                pl.BlockSpec((r, 1), lambda i: (i, 0)),
                pl.BlockSpec((d, dout), lambda i: (0, 0)),
                pl.BlockSpec((d, dout), lambda i: (0, 0)),
                pl.BlockSpec((d, dout), lambda i: (0, 0)),
                pl.BlockSpec((1, dout), lambda i: (0, 0))]
    if final:
        return pl.pallas_call(
            _final_body,
            grid=(n // r,),
            in_specs=in_specs,
            out_specs=pl.BlockSpec((r, dout), lambda i: (i, 0)),
            out_shape=jax.ShapeDtypeStruct((n, dout), jnp.float32),
        )
    return pl.pallas_call(
        _layer_body,
        grid=(n // r,),
        in_specs=in_specs,
        out_specs=[pl.BlockSpec((r, dout), lambda i: (i, 0)),
                   pl.BlockSpec((r, dout), lambda i: (i, 0))],
        out_shape=[jax.ShapeDtypeStruct((n, dout), jnp.float32),
                   jax.ShapeDtypeStruct((n, dout), jnp.float32)],
    )


def kernel(x, edge_index, W_in, b_in, W_h, b_h, W_out, b_out):
    n, d = x.shape
    e = edge_index.shape[1]
    nw = _NC * _NS
    npad = -(-n // 128) * 128
    rt = npad // _NS
    src3 = edge_index[0].reshape(nw, e // nw // 80, 80)
    src4 = edge_index[0].reshape(nw, e // nw // 200, 5, 40)
    dst4 = edge_index[1].reshape(nw, e // nw // 200, 5, 40)

    ones1_c = jnp.ones((80,), jnp.float32)
    zeros1_c = jnp.zeros((npad,), jnp.float32)
    zeros64_c = jnp.zeros((64, 128), jnp.float32)
    degp = _make_deg(n, e)(src3, ones1_c, zeros1_c).reshape(_NC, npad, 1)

    dis, g = _make_prep(n, d, 1)(degp, x)

    prop = _make_prop(n, e, d)
    combine = _make_combine(n, d)

    h = x
    for W, bias, final in ((W_in, b_in, False), (W_h, b_h, False),
                           (W_out, b_out, True)):
        dout = W.shape[2]
        a_w = W[0] - W[2]
        b_w = W[1]
        c_w = -2.0 * W[2]
        p = prop(g, src4, dst4, zeros64_c)
        t1, g2 = combine(p, dis)
        q = prop(g2, src4, dst4, zeros64_c)
        layer = _make_layer(n, d, dout, final)
        res = layer(h, t1, q, dis, a_w, b_w, c_w, bias.reshape(1, dout))
        if final:
            return res
        h, g = res
